# Initial kernel scaffold; baseline (speedup 1.0000x reference)
import jax
import jax.numpy as jnp
from jax.experimental import pallas as pl


def kernel(V, V_velocity, V_w, V_force, V_compliance, C_dist, C_init_d):
    raise NotImplementedError("write your pallas kernel here")


# speedup vs baseline: 13.5721x; 13.5721x over previous
import functools

import jax
import jax.numpy as jnp
from jax import lax
from jax.experimental import pallas as pl
from jax.experimental.pallas import tpu as pltpu
from jax.experimental.pallas import tpu_sc as plsc

N_NODES = 50000
N_EDGES = 1600000
DT = 0.01
ITERATION = 10

NW = 16
NPAD = 50176
ROWS_W = NPAD // NW
CH = 2048
EPW = 100352
NCH = EPW // CH
EPAD = EPW * NW

_MAGIC = 0x5F3759DF


def _body(x_h, y_h, z_h, vx_h, vy_h, vz_h, fx_h, fy_h, fz_h, wn_h, cn_h,
          i1_h, j1_h, d0_h,
          px_h, py_h, pz_h, ux_h, uy_h, uz_h, k_h, ak_h, wi_h, wj_h, l_h,
          ax_sh, ay_sh, az_sh,
          ii_v, jj_v, k_v, ak_v, wi_v, wj_v, d0_v, l_v, tmp_v,
          gxi, gyi, gzi, gxj, gyj, gzj,
          dxi, dyi, dzi, dxj, dyj, dzj,
          buf_a, buf_b, buf_c, buf_w):
    sid = lax.axis_index("s")
    wid = sid
    f32 = jnp.float32
    dt = f32(DT)
    dt2 = f32(DT * DT)
    zero16 = jnp.zeros((16,), f32)
    nan16 = zero16 + f32(jnp.nan)

    nsl = pl.ds(wid * ROWS_W, ROWS_W)

    pltpu.sync_copy(wn_h.at[nsl], buf_w)
    for pos_h, vel_h, f_h, p_h, acc_sh in (
            (x_h, vx_h, fx_h, px_h, ax_sh),
            (y_h, vy_h, fy_h, py_h, ay_sh),
            (z_h, vz_h, fz_h, pz_h, az_sh)):
        pltpu.sync_copy(pos_h.at[nsl], buf_a)
        pltpu.sync_copy(vel_h.at[nsl], buf_b)
        pltpu.sync_copy(f_h.at[nsl], buf_c)

        @pl.loop(0, ROWS_W // 16)
        def _pred(t):
            s = pl.ds(t * 16, 16)
            buf_a[s] = buf_a[s] + dt * buf_b[s] + dt2 * buf_w[s] * buf_c[s]

        pltpu.sync_copy(buf_a, p_h.at[nsl])
        pltpu.sync_copy(buf_a, acc_sh.at[nsl])

    @pl.loop(0, CH // 16)
    def _zl(t):
        l_v[pl.ds(t * 16, 16)] = zero16

    @pl.loop(0, NCH)
    def _coef(c):
        sl = pl.ds(wid * EPW + c * CH, CH)
        pltpu.sync_copy(i1_h.at[sl], ii_v)
        pltpu.sync_copy(j1_h.at[sl], jj_v)
        pltpu.sync_copy(wn_h.at[ii_v], wi_v)
        pltpu.sync_copy(wn_h.at[jj_v], wj_v)
        pltpu.sync_copy(cn_h.at[ii_v], d0_v)
        pltpu.sync_copy(cn_h.at[jj_v], tmp_v)

        @pl.loop(0, CH // 16)
        def _ck(t):
            s = pl.ds(t * 16, 16)
            wi = wi_v[s]
            wj = wj_v[s]
            a = f32(0.5) * (d0_v[s] + tmp_v[s])
            ssum = wi + wj
            k = jnp.where(ssum == 0.0, f32(0.0), f32(1.0) / (ssum + a))
            k_v[s] = k
            ak_v[s] = a * k

        pltpu.sync_copy(k_v, k_h.at[sl])
        pltpu.sync_copy(ak_v, ak_h.at[sl])
        pltpu.sync_copy(wi_v, wi_h.at[sl])
        pltpu.sync_copy(wj_v, wj_h.at[sl])
        pltpu.sync_copy(l_v, l_h.at[sl])

    plsc.subcore_barrier()

    @pl.loop(0, ITERATION)
    def _iter(_):
        @pl.loop(0, NCH)
        def _chunk(c):
            sl = pl.ds(wid * EPW + c * CH, CH)
            pltpu.sync_copy(i1_h.at[sl], ii_v)
            pltpu.sync_copy(j1_h.at[sl], jj_v)
            pltpu.sync_copy(k_h.at[sl], k_v)
            pltpu.sync_copy(ak_h.at[sl], ak_v)
            pltpu.sync_copy(wi_h.at[sl], wi_v)
            pltpu.sync_copy(wj_h.at[sl], wj_v)
            pltpu.sync_copy(d0_h.at[sl], d0_v)
            pltpu.sync_copy(l_h.at[sl], l_v)
            pltpu.sync_copy(px_h.at[ii_v], gxi)
            pltpu.sync_copy(py_h.at[ii_v], gyi)
            pltpu.sync_copy(pz_h.at[ii_v], gzi)
            pltpu.sync_copy(px_h.at[jj_v], gxj)
            pltpu.sync_copy(py_h.at[jj_v], gyj)
            pltpu.sync_copy(pz_h.at[jj_v], gzj)

            @pl.loop(0, CH // 16)
            def _edge(t):
                s = pl.ds(t * 16, 16)
                dx = gxi[s] - gxj[s]
                dy = gyi[s] - gyj[s]
                dz = gzi[s] - gzj[s]
                d2 = dx * dx + dy * dy + dz * dz
                bits = plsc.bitcast(d2, jnp.int32)
                y = plsc.bitcast(_MAGIC - (bits >> 1), f32)
                hd = f32(0.5) * d2
                y = y * (f32(1.5) - hd * y * y)
                y = y * (f32(1.5) - hd * y * y)
                dnorm = d2 * y
                invd = jnp.where(d2 == 0.0, nan16, y)
                el = l_v[s]
                ld = (d0_v[s] - dnorm) * k_v[s] - ak_v[s] * el
                l_v[s] = el + ld
                g = ld * invd
                ai = wi_v[s] * g
                aj = -(wj_v[s] * g)
                dxi[s] = ai * dx
                dyi[s] = ai * dy
                dzi[s] = ai * dz
                dxj[s] = aj * dx
                dyj[s] = aj * dy
                dzj[s] = aj * dz

            pltpu.sync_copy(l_v, l_h.at[sl])
            pltpu.sync_copy(dxi, ax_sh.at[ii_v], add=True)
            pltpu.sync_copy(dyi, ay_sh.at[ii_v], add=True)
            pltpu.sync_copy(dzi, az_sh.at[ii_v], add=True)
            pltpu.sync_copy(dxj, ax_sh.at[jj_v], add=True)
            pltpu.sync_copy(dyj, ay_sh.at[jj_v], add=True)
            pltpu.sync_copy(dzj, az_sh.at[jj_v], add=True)

        plsc.subcore_barrier()
        for acc_sh, p_h in ((ax_sh, px_h), (ay_sh, py_h), (az_sh, pz_h)):
            pltpu.sync_copy(acc_sh.at[nsl], buf_a)
            pltpu.sync_copy(buf_a, p_h.at[nsl])
        plsc.subcore_barrier()

    for p_h, pos_h, u_h in ((px_h, x_h, ux_h), (py_h, y_h, uy_h),
                            (pz_h, z_h, uz_h)):
        pltpu.sync_copy(p_h.at[nsl], buf_a)
        pltpu.sync_copy(pos_h.at[nsl], buf_b)

        @pl.loop(0, ROWS_W // 16)
        def _vel(t):
            s = pl.ds(t * 16, 16)
            buf_b[s] = (buf_a[s] - buf_b[s]) / dt

        pltpu.sync_copy(buf_b, u_h.at[nsl])


@jax.jit
def _xpbd(x, y, z, vx, vy, vz, fx, fy, fz, wn, cn, i1, j1, d0):
    f32 = jnp.float32
    mesh = plsc.VectorSubcoreMesh(core_axis_name="c", subcore_axis_name="s",
                                  num_cores=1)
    node = jax.ShapeDtypeStruct((NPAD,), f32)
    edge = jax.ShapeDtypeStruct((EPAD,), f32)
    out_type = (node, node, node,
                node, node, node,
                edge, edge, edge, edge, edge)
    evmem = pltpu.VMEM((CH,), f32)
    scratch = [
        pltpu.VMEM_SHARED((NPAD,), f32),
        pltpu.VMEM_SHARED((NPAD,), f32),
        pltpu.VMEM_SHARED((NPAD,), f32),
        pltpu.VMEM((CH,), jnp.int32),
        pltpu.VMEM((CH,), jnp.int32),
        evmem, evmem, evmem, evmem, evmem, evmem, evmem,
        evmem, evmem, evmem, evmem, evmem, evmem,
        evmem, evmem, evmem, evmem, evmem, evmem,
        pltpu.VMEM((ROWS_W,), f32),
        pltpu.VMEM((ROWS_W,), f32),
        pltpu.VMEM((ROWS_W,), f32),
        pltpu.VMEM((ROWS_W,), f32),
    ]
    fn = pl.kernel(_body, out_type=out_type, mesh=mesh, scratch_types=scratch,
                   compiler_params=pltpu.CompilerParams(
                       needs_layout_passes=False,
                       use_tc_tiling_on_sc=False))
    return fn(x, y, z, vx, vy, vz, fx, fy, fz, wn, cn, i1, j1, d0)


def kernel(V, V_velocity, V_w, V_force, V_compliance, C_dist, C_init_d):
    f32 = jnp.float32
    n = V.shape[0]
    e = C_dist.shape[0]

    def padn(a, tail=0.0):
        return jnp.full((NPAD,), f32(tail)).at[:n].set(a.astype(f32))

    V = V.astype(f32)
    x = padn(V[:, 0])
    x = x.at[n:].set(jnp.float32(1.0) + jnp.arange(NPAD - n, dtype=f32))
    y = padn(V[:, 1])
    z = padn(V[:, 2])
    vx = padn(V_velocity[:, 0])
    vy = padn(V_velocity[:, 1])
    vz = padn(V_velocity[:, 2])
    fx = padn(V_force[:, 0])
    fy = padn(V_force[:, 1])
    fz = padn(V_force[:, 2])
    wn = padn(V_w[:, 0])
    cn = padn(V_compliance[:, 0])
    i1 = jnp.full((EPAD,), n, jnp.int32).at[:e].set(C_dist[:, 0].astype(jnp.int32))
    j1 = jnp.full((EPAD,), n + 1, jnp.int32).at[:e].set(C_dist[:, 1].astype(jnp.int32))
    d0 = jnp.ones((EPAD,), f32).at[:e].set(C_init_d[:, 0].astype(f32))

    px, py, pz, ux, uy, uz, *_ = _xpbd(x, y, z, vx, vy, vz, fx, fy, fz,
                                       wn, cn, i1, j1, d0)
    Vout = jnp.stack([px[:n], py[:n], pz[:n]], axis=1)
    Velout = jnp.stack([ux[:n], uy[:n], uz[:n]], axis=1)
    return Vout, Velout
